# baseline (speedup 1.0000x reference)
import functools

import jax
import jax.numpy as jnp
from jax import lax
from jax.experimental import pallas as pl
from jax.experimental.pallas import tpu as pltpu
from jax.experimental.pallas import tpu_sc as plsc

_RADIUS = 32
_NC = 2
_NS = 16
_FIRE = 8


def _sc_body(wf_hbm, out_hbm, w_v, strip_v, sem, *, L, B, K, D):
    wid = lax.axis_index("c") * _NS + lax.axis_index("s")
    k = lax.rem(wid, 4)
    t8 = lax.div(wid, 4)
    pltpu.sync_copy(wf_hbm, w_v)

    v0a = w_v[pl.ds(0, 16)]
    v0b = w_v[pl.ds(16, 16)]
    vka = w_v[pl.ds((K - 1) * D, 16)]
    vkb = w_v[pl.ds((K - 1) * D + 16, 16)]
    lo_slot = (L - 1 - _RADIUS)

    def _fill(lo, hi, va, vb):
        def body(s, c):
            for j in range(8):
                strip_v[s, pl.ds(16 * j, 16)] = va if j % 2 == 0 else vb
            return c

        lax.fori_loop(lo, hi, body, 0)

    _fill(0, 118, v0a, v0b)
    _fill(138, 2 * L // 4, vka, vkb)

    def _band(s, c):
        for j in range(8):
            t = 4 * s + k + j // 2
            idx = lax.clamp(0, t - lo_slot, K - 1)
            strip_v[s, pl.ds(16 * j, 16)] = w_v[pl.ds(idx * D + (j % 2) * 16, 16)]
        return c

    lax.fori_loop(118, 138, _band, 0)

    c_res = 3 - k

    def _grp(g, carry):
        cps = []
        for t in range(_FIRE):
            m = t8 * (B * L // 32) + g * _FIRE + t
            b = lax.div(m, L // 4)
            ii = lax.rem(m, L // 4)
            i = 4 * ii + c_res
            srow = (L // 4 - 1) - ii
            cps.append(
                pltpu.async_copy(
                    strip_v.at[pl.ds(srow, L * D // 128), :],
                    out_hbm.at[b, i],
                    sem,
                )
            )
        for cp in cps:
            cp.wait()
        return carry

    lax.fori_loop(0, (B * L // 32) // _FIRE, _grp, 0)


def kernel(x, W):
    L, B = x.shape[0], x.shape[1]
    K, D = W.shape
    mesh = plsc.VectorSubcoreMesh(
        core_axis_name="c", subcore_axis_name="s", num_cores=_NC, num_subcores=_NS
    )
    sc_call = functools.partial(
        pl.kernel,
        out_type=jax.ShapeDtypeStruct((B, L, L * D // 128, 128), jnp.float32),
        mesh=mesh,
        scratch_types=[
            pltpu.VMEM((K * D,), jnp.float32),
            pltpu.VMEM((2 * L // 4, 128), jnp.float32),
            pltpu.SemaphoreType.DMA,
        ],
    )(functools.partial(_sc_body, L=L, B=B, K=K, D=D))
    out = sc_call(W.reshape(-1))
    return out.reshape(B, L, L, D)
